# baseline (speedup 1.0000x reference)
import functools

import jax
import jax.numpy as jnp
from jax import lax
from jax.experimental import pallas as pl
from jax.experimental.pallas import tpu as pltpu
from jax.experimental.pallas import tpu_sc as plsc

NUM_FIELDS = 26
VOCAB = 100000
EMB_DIM = 50
BATCH = 16384
OUT_D = NUM_FIELDS * EMB_DIM

NC = 2
NS = 16
L = 16
NW = NC * NS

W = 32
NWIN = 3
TOTAL_WORDS = NUM_FIELDS * VOCAB * EMB_DIM
VROWS = TOTAL_WORDS // W

CHB = 4
CHR = CHB * NUM_FIELDS
PADR = 112
NGRP = PADR // L
BPW = BATCH // NW
NCH = BPW // CHB


def _make_gather():
  mesh = plsc.VectorSubcoreMesh(core_axis_name="c", subcore_axis_name="s")

  @functools.partial(
      pl.kernel,
      out_type=jax.ShapeDtypeStruct((BATCH * NUM_FIELDS, EMB_DIM), jnp.float32),
      mesh=mesh,
      scratch_types=[
          pltpu.VMEM((BPW, NUM_FIELDS), jnp.int32),
          pltpu.VMEM((PADR,), jnp.int32),
          pltpu.VMEM((PADR,), jnp.int32),
          pltpu.VMEM((PADR,), jnp.int32),
          pltpu.VMEM((PADR,), jnp.int32),
          pltpu.VMEM((PADR,), jnp.int32),
          pltpu.VMEM((PADR,), jnp.int32),
          pltpu.VMEM((PADR,), jnp.int32),
          pltpu.VMEM((PADR,), jnp.int32),
          pltpu.VMEM((NWIN, PADR, W), jnp.float32),
          pltpu.VMEM((NWIN, PADR, W), jnp.float32),
          pltpu.VMEM((CHR, EMB_DIM), jnp.float32),
          pltpu.VMEM((CHR, EMB_DIM), jnp.float32),
          pltpu.SemaphoreType.DMA,
          pltpu.SemaphoreType.DMA,
          pltpu.SemaphoreType.DMA,
          pltpu.SemaphoreType.DMA,
      ],
      compiler_params=pltpu.CompilerParams(
          use_tc_tiling_on_sc=False, needs_layout_passes=False),
  )
  def grab(cat_hbm, table_hbm, out_hbm, catv, aqa0, aqa1, aqb0, aqb1,
           aqc0, aqc1, of0, of1, win0, win1, ob0, ob1,
           gs0, gs1, os0, os1):
    aqa = (aqa0, aqa1)
    aqb = (aqb0, aqb1)
    aqc = (aqc0, aqc1)
    off = (of0, of1)
    win = (win0, win1)
    outb = (ob0, ob1)
    gsem = (gs0, gs1)
    osem = (os0, os1)

    wid = lax.axis_index("s") * NC + lax.axis_index("c")
    b_base = wid * BPW

    def lanevecs(s):
      lane = lax.iota(jnp.int32, L)
      jv = s * L + lane
      if (s + 1) * L > CHR:
        msk = lane < (CHR - s * L)
        jsafe = jnp.where(msk, jv, 0)
      else:
        msk = None
        jsafe = jv
      rowv = lax.shift_right_logical(jsafe * 2521, 16)
      fldv = jsafe - rowv * NUM_FIELDS
      return rowv, fldv, jv, fldv * EMB_DIM, msk

    def build(c, slot):
      for s in range(NGRP):
        rowv, fldv, _, _, msk = lanevecs(s)
        cvals = plsc.load_gather(catv, [c * CHB + rowv, fldv])
        t = cvals + fldv * VOCAB
        w0 = t * EMB_DIM
        a = lax.shift_right_logical(w0, 5)
        o = lax.bitwise_and(w0, W - 1)
        if msk is not None:
          a = jnp.where(msk, a, 0)
          o = jnp.where(msk, o, 0)
        off[slot][pl.ds(s * L, L)] = o
        aqa[slot][pl.ds(s * L, L)] = a
        aqb[slot][pl.ds(s * L, L)] = a + 1
        aqc[slot][pl.ds(s * L, L)] = a + 2

    def gathers(slot):
      return [
          pltpu.make_async_copy(
              table_hbm.at[aqa[slot]], win[slot].at[0], gsem[slot]),
          pltpu.make_async_copy(
              table_hbm.at[aqb[slot]], win[slot].at[1], gsem[slot]),
          pltpu.make_async_copy(
              table_hbm.at[aqc[slot]], win[slot].at[2], gsem[slot]),
      ]

    def out_copy(c, slot):
      return pltpu.make_async_copy(
          outb[slot],
          out_hbm.at[pl.ds((b_base + c * CHB) * NUM_FIELDS, CHR)],
          osem[slot])

    def realign(slot):
      for s in range(NGRP):
        rowv, _, jwin, colv, msk = lanevecs(s)
        offv = off[slot][pl.ds(s * L, L)]
        for k in range(EMB_DIM):
          g = offv + k
          qv = lax.shift_right_logical(g, 5)
          cv = lax.bitwise_and(g, W - 1)
          val = plsc.load_gather(win[slot], [qv, jwin, cv])
          plsc.store_scatter(outb[slot], [jwin, jnp.full((L,), k, jnp.int32)],
                             val, mask=msk)

    def step(c, slot):
      other = 1 - slot

      for cp in gathers(slot):
        cp.wait()

      @pl.when(c + 1 < NCH)
      def _():
        for cp in gathers(other):
          cp.start()

      @pl.when(c >= 1)
      def _():
        out_copy(c - 1, other).start()

      @pl.when(c >= 2)
      def _():
        out_copy(c - 2, slot).wait()

      realign(slot)

      @pl.when(c + 2 < NCH)
      def _():
        build(c + 2, slot)

    pltpu.sync_copy(cat_hbm.at[pl.ds(b_base, BPW)], catv)
    build(0, 0)
    build(1, 1)
    plsc.subcore_barrier()
    for cp in gathers(0):
      cp.start()

    def loop_body(c2, carry):
      step(2 * c2, 0)
      step(2 * c2 + 1, 1)
      return carry

    lax.fori_loop(0, NCH // 2, loop_body, 0)

    plsc.subcore_barrier()
    out_copy(NCH - 1, 1).start()
    out_copy(NCH - 2, 0).wait()
    out_copy(NCH - 1, 1).wait()

  return grab


_gather = _make_gather()


def kernel(cat, tables):
  table32 = tables.reshape(VROWS, W)
  out = _gather(cat, table32)
  return out.reshape(BATCH, OUT_D)


# speedup vs baseline: 1.0849x; 1.0849x over previous
import functools

import jax
import jax.numpy as jnp
from jax import lax
from jax.experimental import pallas as pl
from jax.experimental.pallas import tpu as pltpu
from jax.experimental.pallas import tpu_sc as plsc

NUM_FIELDS = 26
VOCAB = 100000
EMB_DIM = 50
BATCH = 16384
OUT_D = NUM_FIELDS * EMB_DIM

NC = 2
NS = 16
L = 16
NW = NC * NS

W = 32
NWIN = 3
TOTAL_WORDS = NUM_FIELDS * VOCAB * EMB_DIM
VROWS = TOTAL_WORDS // W

CHB = 4
CHR = CHB * NUM_FIELDS
PADR = 112
NGRP = PADR // L
BPW = BATCH // NW
NCH = BPW // CHB


def _make_gather():
  mesh = plsc.VectorSubcoreMesh(core_axis_name="c", subcore_axis_name="s")

  @functools.partial(
      pl.kernel,
      out_type=jax.ShapeDtypeStruct((BATCH, OUT_D), jnp.float32),
      mesh=mesh,
      scratch_types=[
          pltpu.VMEM((BPW, NUM_FIELDS), jnp.int32),
          pltpu.VMEM((PADR,), jnp.int32),
          pltpu.VMEM((PADR,), jnp.int32),
          pltpu.VMEM((PADR,), jnp.int32),
          pltpu.VMEM((PADR,), jnp.int32),
          pltpu.VMEM((PADR,), jnp.int32),
          pltpu.VMEM((PADR,), jnp.int32),
          pltpu.VMEM((PADR,), jnp.int32),
          pltpu.VMEM((PADR,), jnp.int32),
          pltpu.VMEM((NWIN, PADR, W), jnp.float32),
          pltpu.VMEM((NWIN, PADR, W), jnp.float32),
          pltpu.VMEM((CHB, OUT_D), jnp.float32),
          pltpu.VMEM((CHB, OUT_D), jnp.float32),
          pltpu.SemaphoreType.DMA,
          pltpu.SemaphoreType.DMA,
          pltpu.SemaphoreType.DMA,
          pltpu.SemaphoreType.DMA,
      ],
      compiler_params=pltpu.CompilerParams(
          use_tc_tiling_on_sc=False, needs_layout_passes=False),
  )
  def grab(cat_hbm, table_hbm, out_hbm, catv, aqa0, aqa1, aqb0, aqb1,
           aqc0, aqc1, of0, of1, win0, win1, ob0, ob1,
           gs0, gs1, os0, os1):
    aqa = (aqa0, aqa1)
    aqb = (aqb0, aqb1)
    aqc = (aqc0, aqc1)
    off = (of0, of1)
    win = (win0, win1)
    outb = (ob0, ob1)
    gsem = (gs0, gs1)
    osem = (os0, os1)

    wid = lax.axis_index("s") * NC + lax.axis_index("c")
    b_base = wid * BPW

    def lanevecs(s):
      lane = lax.iota(jnp.int32, L)
      jv = s * L + lane
      if (s + 1) * L > CHR:
        msk = lane < (CHR - s * L)
        jsafe = jnp.where(msk, jv, 0)
      else:
        msk = None
        jsafe = jv
      rowv = lax.shift_right_logical(jsafe * 2521, 16)
      fldv = jsafe - rowv * NUM_FIELDS
      return rowv, fldv, jv, fldv * EMB_DIM, msk

    def build(c, slot):
      for s in range(NGRP):
        rowv, fldv, _, _, msk = lanevecs(s)
        cvals = plsc.load_gather(catv, [c * CHB + rowv, fldv])
        t = cvals + fldv * VOCAB
        w0 = t * EMB_DIM
        a = lax.shift_right_logical(w0, 5)
        o = lax.bitwise_and(w0, W - 1)
        if msk is not None:
          a = jnp.where(msk, a, 0)
          o = jnp.where(msk, o, 0)
        off[slot][pl.ds(s * L, L)] = o
        aqa[slot][pl.ds(s * L, L)] = a
        aqb[slot][pl.ds(s * L, L)] = a + 1
        aqc[slot][pl.ds(s * L, L)] = a + 2

    def gathers(slot):
      return [
          pltpu.make_async_copy(
              table_hbm.at[aqa[slot]], win[slot].at[0], gsem[slot]),
          pltpu.make_async_copy(
              table_hbm.at[aqb[slot]], win[slot].at[1], gsem[slot]),
          pltpu.make_async_copy(
              table_hbm.at[aqc[slot]], win[slot].at[2], gsem[slot]),
      ]

    def out_copy(c, slot):
      return pltpu.make_async_copy(
          outb[slot], out_hbm.at[pl.ds(b_base + c * CHB, CHB)], osem[slot])

    def realign(slot):
      for s in range(NGRP):
        rowv, _, jwin, colv, msk = lanevecs(s)
        offv = off[slot][pl.ds(s * L, L)]
        for k in range(EMB_DIM):
          g = offv + k
          qv = lax.shift_right_logical(g, 5)
          cv = lax.bitwise_and(g, W - 1)
          val = plsc.load_gather(win[slot], [qv, jwin, cv])
          plsc.store_scatter(outb[slot], [rowv, colv + k], val, mask=msk)

    def step(c, slot):
      other = 1 - slot

      @pl.when(c + 1 < NCH)
      def _():
        for cp in gathers(other):
          cp.start()

      for cp in gathers(slot):
        cp.wait()

      @pl.when(c >= 1)
      def _():
        out_copy(c - 1, other).start()

      @pl.when(c >= 2)
      def _():
        out_copy(c - 2, slot).wait()

      realign(slot)

      @pl.when(c + 2 < NCH)
      def _():
        build(c + 2, slot)

    pltpu.sync_copy(cat_hbm.at[pl.ds(b_base, BPW)], catv)
    build(0, 0)
    build(1, 1)
    plsc.subcore_barrier()
    for cp in gathers(0):
      cp.start()

    def loop_body(c2, carry):
      step(2 * c2, 0)
      step(2 * c2 + 1, 1)
      return carry

    lax.fori_loop(0, NCH // 2, loop_body, 0)

    plsc.subcore_barrier()
    out_copy(NCH - 1, 1).start()
    out_copy(NCH - 2, 0).wait()
    out_copy(NCH - 1, 1).wait()

  return grab


_gather = _make_gather()


def kernel(cat, tables):
  table32 = tables.reshape(VROWS, W)
  return _gather(cat, table32)
